# baseline (device time: 85634 ns/iter reference)
import jax
import jax.numpy as jnp
from jax import lax
from jax.experimental import pallas as pl
from jax.experimental.pallas import tpu as pltpu

N_Z = 4


def kernel(partial, resid, gamma):
    _, m, d = partial.shape
    gamma2 = gamma.reshape(1, d)

    def body(p_ref, r_ref, g_ref, out_ref, comm_ref, send_sems, recv_sems):
        my_x = lax.axis_index("x")
        my_y = lax.axis_index("y")
        my_z = lax.axis_index("z")
        left = (my_z - 1) % N_Z
        right = (my_z + 1) % N_Z

        barrier_sem = pltpu.get_barrier_semaphore()
        for nbr in [left, right]:
            pl.semaphore_signal(
                barrier_sem,
                inc=1,
                device_id=(my_x, my_y, nbr),
                device_id_type=pl.DeviceIdType.MESH,
            )
        pl.semaphore_wait(barrier_sem, 2)

        comm_ref[0] = p_ref[0].astype(jnp.bfloat16)
        acc = p_ref[0]

        for h in range(N_Z - 1):
            rdma = pltpu.make_async_remote_copy(
                src_ref=comm_ref.at[h],
                dst_ref=comm_ref.at[h + 1],
                send_sem=send_sems.at[h],
                recv_sem=recv_sems.at[h],
                device_id=(my_x, my_y, right),
                device_id_type=pl.DeviceIdType.MESH,
            )
            rdma.start()
            rdma.wait()
            acc = acc + comm_ref[h + 1].astype(jnp.float32)

        y = acc + r_ref[...]
        rms = jnp.sqrt(jnp.mean(y * y, axis=-1, keepdims=True) + 1e-6)
        out_ref[...] = (y / rms) * g_ref[...]

    return pl.pallas_call(
        body,
        out_shape=jax.ShapeDtypeStruct((m, d), jnp.float32),
        in_specs=[
            pl.BlockSpec(memory_space=pltpu.VMEM),
            pl.BlockSpec(memory_space=pltpu.VMEM),
            pl.BlockSpec(memory_space=pltpu.VMEM),
        ],
        out_specs=pl.BlockSpec(memory_space=pltpu.VMEM),
        scratch_shapes=[
            pltpu.VMEM((N_Z, m, d), jnp.bfloat16),
            pltpu.SemaphoreType.DMA((N_Z - 1,)),
            pltpu.SemaphoreType.DMA((N_Z - 1,)),
        ],
        compiler_params=pltpu.CompilerParams(collective_id=0),
    )(partial, resid, gamma2)


# device time: 56743 ns/iter; 1.5092x vs baseline; 1.5092x over previous
import jax
import jax.numpy as jnp
from jax import lax
from jax.experimental import pallas as pl
from jax.experimental.pallas import tpu as pltpu

N_Z = 4


def kernel(partial, resid, gamma):
    _, m, d = partial.shape
    mc = m // N_Z
    gamma2 = gamma.reshape(1, d)

    def body(p_ref, r_ref, g_ref, out_ref,
             rs_send, rs_recv, ag_buf,
             rs_send_sems, rs_recv_sems, ag_send_sems, ag_recv_sems):
        my_x = lax.axis_index("x")
        my_y = lax.axis_index("y")
        my_z = lax.axis_index("z")
        left = (my_z - 1) % N_Z
        right = (my_z + 1) % N_Z

        barrier_sem = pltpu.get_barrier_semaphore()
        for nbr in [left, right]:
            pl.semaphore_signal(
                barrier_sem,
                inc=1,
                device_id=(my_x, my_y, nbr),
                device_id_type=pl.DeviceIdType.MESH,
            )
        pl.semaphore_wait(barrier_sem, 2)

        def my_chunk(c):
            return p_ref[0, pl.ds(c * mc, mc), :]

        for s in range(N_Z - 1):
            send_c = (my_z - s) % N_Z
            if s == 0:
                val = my_chunk(send_c).astype(jnp.bfloat16)
            else:
                val = (rs_recv[s - 1].astype(jnp.float32)
                       + my_chunk(send_c)).astype(jnp.bfloat16)
            rs_send[s] = val
            rdma = pltpu.make_async_remote_copy(
                src_ref=rs_send.at[s],
                dst_ref=rs_recv.at[s],
                send_sem=rs_send_sems.at[s],
                recv_sem=rs_recv_sems.at[s],
                device_id=(my_x, my_y, right),
                device_id_type=pl.DeviceIdType.MESH,
            )
            rdma.start()
            rdma.wait()

        own = (my_z + 1) % N_Z
        total = rs_recv[N_Z - 2].astype(jnp.float32) + my_chunk(own)

        y = total + r_ref[pl.ds(own * mc, mc), :]
        rms = jnp.sqrt(jnp.mean(y * y, axis=-1, keepdims=True) + 1e-6)
        norm = (y / rms) * g_ref[...]
        out_ref[pl.ds(own * mc, mc), :] = norm
        ag_buf[0] = norm.astype(jnp.bfloat16)

        for t in range(N_Z - 1):
            rdma = pltpu.make_async_remote_copy(
                src_ref=ag_buf.at[t],
                dst_ref=ag_buf.at[t + 1],
                send_sem=ag_send_sems.at[t],
                recv_sem=ag_recv_sems.at[t],
                device_id=(my_x, my_y, right),
                device_id_type=pl.DeviceIdType.MESH,
            )
            rdma.start()
            rdma.wait()
            origin = (own - t - 1) % N_Z
            out_ref[pl.ds(origin * mc, mc), :] = ag_buf[t + 1].astype(jnp.float32)

    return pl.pallas_call(
        body,
        out_shape=jax.ShapeDtypeStruct((m, d), jnp.float32),
        in_specs=[
            pl.BlockSpec(memory_space=pltpu.VMEM),
            pl.BlockSpec(memory_space=pltpu.VMEM),
            pl.BlockSpec(memory_space=pltpu.VMEM),
        ],
        out_specs=pl.BlockSpec(memory_space=pltpu.VMEM),
        scratch_shapes=[
            pltpu.VMEM((N_Z - 1, mc, d), jnp.bfloat16),
            pltpu.VMEM((N_Z - 1, mc, d), jnp.bfloat16),
            pltpu.VMEM((N_Z, mc, d), jnp.bfloat16),
            pltpu.SemaphoreType.DMA((N_Z - 1,)),
            pltpu.SemaphoreType.DMA((N_Z - 1,)),
            pltpu.SemaphoreType.DMA((N_Z - 1,)),
            pltpu.SemaphoreType.DMA((N_Z - 1,)),
        ],
        compiler_params=pltpu.CompilerParams(collective_id=0),
    )(partial, resid, gamma2)


# device time: 43672 ns/iter; 1.9608x vs baseline; 1.2993x over previous
import jax
import jax.numpy as jnp
from jax import lax
from jax.experimental import pallas as pl
from jax.experimental.pallas import tpu as pltpu

N_Z = 4


def kernel(partial, resid, gamma):
    _, m, d = partial.shape
    mh = m // 2
    mc = mh // N_Z
    gamma2 = gamma.reshape(1, d)

    def body(p_ref, r_ref, g_ref, out_ref,
             rs_send, rs_recv, ag_buf, xg_buf,
             rs_send_sems, rs_recv_sems, ag_send_sems, ag_recv_sems,
             x_send_sems, x_recv_sems):
        my_x = lax.axis_index("x")
        my_y = lax.axis_index("y")
        my_z = lax.axis_index("z")
        left = (my_z - 1) % N_Z
        right = (my_z + 1) % N_Z
        xn = 1 - my_x
        my_base = my_x * mh
        nb_base = xn * mh

        barrier_sem = pltpu.get_barrier_semaphore()
        for dev in [(my_x, my_y, left), (my_x, my_y, right), (xn, my_y, my_z)]:
            pl.semaphore_signal(
                barrier_sem, inc=1,
                device_id=dev, device_id_type=pl.DeviceIdType.MESH,
            )
        pl.semaphore_wait(barrier_sem, 3)

        def my_chunk(c):
            return p_ref[0, pl.ds(my_base + c * mc, mc), :]

        for s in range(N_Z - 1):
            send_c = (my_z - s) % N_Z
            if s == 0:
                val = my_chunk(send_c).astype(jnp.bfloat16)
            else:
                val = (rs_recv[s - 1].astype(jnp.float32)
                       + my_chunk(send_c)).astype(jnp.bfloat16)
            rs_send[s] = val
            rdma = pltpu.make_async_remote_copy(
                src_ref=rs_send.at[s],
                dst_ref=rs_recv.at[s],
                send_sem=rs_send_sems.at[s],
                recv_sem=rs_recv_sems.at[s],
                device_id=(my_x, my_y, right),
                device_id_type=pl.DeviceIdType.MESH,
            )
            rdma.start()
            rdma.wait()

        own = (my_z + 1) % N_Z
        total = rs_recv[N_Z - 2].astype(jnp.float32) + my_chunk(own)

        y = total + r_ref[pl.ds(my_base + own * mc, mc), :]
        rms = jnp.sqrt(jnp.mean(y * y, axis=-1, keepdims=True) + 1e-6)
        norm = (y / rms) * g_ref[...]
        ag_buf[0] = norm.astype(jnp.bfloat16)
        out_ref[pl.ds(my_base + own * mc, mc), :] = norm

        def x_send(k):
            rdma = pltpu.make_async_remote_copy(
                src_ref=ag_buf.at[k],
                dst_ref=xg_buf.at[k],
                send_sem=x_send_sems.at[k],
                recv_sem=x_recv_sems.at[k],
                device_id=(xn, my_y, my_z),
                device_id_type=pl.DeviceIdType.MESH,
            )
            rdma.start()
            return rdma

        x_rdmas = [x_send(0)]

        for t in range(N_Z - 1):
            rdma = pltpu.make_async_remote_copy(
                src_ref=ag_buf.at[t],
                dst_ref=ag_buf.at[t + 1],
                send_sem=ag_send_sems.at[t],
                recv_sem=ag_recv_sems.at[t],
                device_id=(my_x, my_y, right),
                device_id_type=pl.DeviceIdType.MESH,
            )
            rdma.start()
            rdma.wait()
            x_rdmas.append(x_send(t + 1))
            origin = (own - t - 1) % N_Z
            out_ref[pl.ds(my_base + origin * mc, mc), :] = (
                ag_buf[t + 1].astype(jnp.float32))

        for k in range(N_Z):
            x_rdmas[k].wait_recv()
            origin = (own - k) % N_Z
            out_ref[pl.ds(nb_base + origin * mc, mc), :] = (
                xg_buf[k].astype(jnp.float32))
        for k in range(N_Z):
            x_rdmas[k].wait_send()

    return pl.pallas_call(
        body,
        out_shape=jax.ShapeDtypeStruct((m, d), jnp.float32),
        in_specs=[
            pl.BlockSpec(memory_space=pltpu.VMEM),
            pl.BlockSpec(memory_space=pltpu.VMEM),
            pl.BlockSpec(memory_space=pltpu.VMEM),
        ],
        out_specs=pl.BlockSpec(memory_space=pltpu.VMEM),
        scratch_shapes=[
            pltpu.VMEM((N_Z - 1, mc, d), jnp.bfloat16),
            pltpu.VMEM((N_Z - 1, mc, d), jnp.bfloat16),
            pltpu.VMEM((N_Z, mc, d), jnp.bfloat16),
            pltpu.VMEM((N_Z, mc, d), jnp.bfloat16),
            pltpu.SemaphoreType.DMA((N_Z - 1,)),
            pltpu.SemaphoreType.DMA((N_Z - 1,)),
            pltpu.SemaphoreType.DMA((N_Z - 1,)),
            pltpu.SemaphoreType.DMA((N_Z - 1,)),
            pltpu.SemaphoreType.DMA((N_Z,)),
            pltpu.SemaphoreType.DMA((N_Z,)),
        ],
        compiler_params=pltpu.CompilerParams(collective_id=0),
    )(partial, resid, gamma2)
